# Initial kernel scaffold; baseline (speedup 1.0000x reference)
import jax
import jax.numpy as jnp
from jax.experimental import pallas as pl


def kernel(x_bnd, params, edge_index, x_atm, x_atm_batch):
    raise NotImplementedError("write your pallas kernel here")


# speedup vs baseline: 2.7155x; 2.7155x over previous
import functools

import jax
import jax.numpy as jnp
from jax import lax
from jax.experimental import pallas as pl
from jax.experimental.pallas import tpu as pltpu
from jax.experimental.pallas import tpu_sc as plsc

N = 10000
E = 320000
D = 128
G = 64
LAYERS = 6
CUT = 6.0

BN = 1000
BE = 2000

NC = 2
NS = 16
NW = NC * NS
EPW = E // NW
EPS = E // NS
CH = 80
DRN = 200
NCHK = N // DRN


def _node_pre_body(xa_ref, emb_ref, a_ref, b_ref, v_ref, u_ref, bias_ref,
                   h_ref, tabd_ref, tabs_ref, hu_ref):
    xa = xa_ref[...]
    e0 = emb_ref[0:1, :]
    e1 = emb_ref[1:2, :]
    e2 = emb_ref[2:3, :]
    h = jnp.where(xa == 0, e0, jnp.where(xa == 1, e1, e2))
    h_ref[...] = h
    tabd_ref[...] = jnp.dot(h, a_ref[...], preferred_element_type=jnp.float32) + bias_ref[0:1, :]
    tabs_ref[:, 0:D] = jnp.dot(h, b_ref[...], preferred_element_type=jnp.float32) + bias_ref[1:2, :]
    tabs_ref[:, D:2 * D] = jnp.dot(h, v_ref[...], preferred_element_type=jnp.float32) + bias_ref[2:3, :]
    hu_ref[...] = jnp.dot(h, u_ref[...], preferred_element_type=jnp.float32) + bias_ref[3:4, :]


def _node_mid_body(h_ref, hu_ref, agg_ref, nrm_ref, a_ref, b_ref, v_ref,
                   u_ref, bias_ref, hn_ref, tabd_ref, tabs_ref, hun_ref):
    upd = hu_ref[...] + agg_ref[...] / (nrm_ref[...] + 1e-6)
    h = h_ref[...] + upd * jax.nn.sigmoid(upd)
    hn_ref[...] = h
    tabd_ref[...] = jnp.dot(h, a_ref[...], preferred_element_type=jnp.float32) + bias_ref[0:1, :]
    tabs_ref[:, 0:D] = jnp.dot(h, b_ref[...], preferred_element_type=jnp.float32) + bias_ref[1:2, :]
    tabs_ref[:, D:2 * D] = jnp.dot(h, v_ref[...], preferred_element_type=jnp.float32) + bias_ref[2:3, :]
    hun_ref[...] = jnp.dot(h, u_ref[...], preferred_element_type=jnp.float32) + bias_ref[3:4, :]


def _wspec(shape):
    return pl.BlockSpec(shape, lambda i: (0,) * len(shape))


def _node_pre(xa2, emb, a, b, v, u, bias):
    out_shape = (
        jax.ShapeDtypeStruct((N, D), jnp.float32),
        jax.ShapeDtypeStruct((N, D), jnp.float32),
        jax.ShapeDtypeStruct((N, 2 * D), jnp.float32),
        jax.ShapeDtypeStruct((N, D), jnp.float32),
    )
    return pl.pallas_call(
        _node_pre_body,
        grid=(N // BN,),
        in_specs=[
            pl.BlockSpec((BN, 1), lambda i: (i, 0)),
            _wspec((8, D)), _wspec((D, D)), _wspec((D, D)), _wspec((D, D)),
            _wspec((D, D)), _wspec((8, D)),
        ],
        out_specs=[
            pl.BlockSpec((BN, D), lambda i: (i, 0)),
            pl.BlockSpec((BN, D), lambda i: (i, 0)),
            pl.BlockSpec((BN, 2 * D), lambda i: (i, 0)),
            pl.BlockSpec((BN, D), lambda i: (i, 0)),
        ],
        out_shape=out_shape,
    )(xa2, emb, a, b, v, u, bias)


def _node_mid(h, hu, agg, nrm, a, b, v, u, bias):
    out_shape = (
        jax.ShapeDtypeStruct((N, D), jnp.float32),
        jax.ShapeDtypeStruct((N, D), jnp.float32),
        jax.ShapeDtypeStruct((N, 2 * D), jnp.float32),
        jax.ShapeDtypeStruct((N, D), jnp.float32),
    )
    nb = pl.BlockSpec((BN, D), lambda i: (i, 0))
    return pl.pallas_call(
        _node_mid_body,
        grid=(N // BN,),
        in_specs=[nb, nb, nb, nb,
                  _wspec((D, D)), _wspec((D, D)), _wspec((D, D)),
                  _wspec((D, D)), _wspec((8, D))],
        out_specs=[nb, nb, pl.BlockSpec((BN, 2 * D), lambda i: (i, 0)), nb],
        out_shape=out_shape,
    )(h, hu, agg, nrm, a, b, v, u, bias)


def _edge_body(first, e_ref, gd_ref, gs_ref, c_ref, bc_ref,
               en_ref, msg_ref, eta_ref):
    if first:
        x = e_ref[...]
        n = lax.broadcasted_iota(jnp.int32, (BE, D), 1).astype(jnp.float32) + 1.0
        e = jnp.sqrt(2.0 / CUT) * jnp.sin(n * (jnp.pi / CUT) * x) / (x + 1e-9)
    else:
        e = e_ref[...]
    gs = gs_ref[...]
    ep = (gd_ref[...] + gs[:, 0:D] + bc_ref[0:1, :]
          + jnp.dot(e, c_ref[...], preferred_element_type=jnp.float32))
    eta = jax.nn.sigmoid(ep)
    en_ref[...] = e + ep * eta
    msg_ref[...] = eta * gs[:, D:2 * D]
    eta_ref[...] = eta


def _edge(first, e_in, gd, gs, c, bc):
    out_shape = (
        jax.ShapeDtypeStruct((E, D), jnp.float32),
        jax.ShapeDtypeStruct((E, D), jnp.float32),
        jax.ShapeDtypeStruct((E, D), jnp.float32),
    )
    eb = pl.BlockSpec((BE, D), lambda i: (i, 0))
    e_spec = pl.BlockSpec((BE, 1), lambda i: (i, 0)) if first else eb
    return pl.pallas_call(
        functools.partial(_edge_body, first),
        grid=(E // BE,),
        in_specs=[e_spec, eb, pl.BlockSpec((BE, 2 * D), lambda i: (i, 0)),
                  _wspec((D, D)), _wspec((8, D))],
        out_specs=[eb, eb, eb],
        out_shape=out_shape,
    )(e_in, gd, gs, c, bc)


def _final_body(h_ref, hu_ref, agg_ref, nrm_ref, b3_ref, wh_ref, bh_ref,
                wo_ref, bo_ref, hout_ref, out_ref, hg_ref):
    i = pl.program_id(0)
    upd = hu_ref[...] + agg_ref[...] / (nrm_ref[...] + 1e-6)
    h = h_ref[...] + upd * jax.nn.sigmoid(upd)
    hout_ref[...] = h
    bids = jnp.reshape(b3_ref[...], (1, BN))
    oh = (lax.broadcasted_iota(jnp.int32, (G, BN), 0) == bids).astype(jnp.float32)
    part = jnp.dot(oh, h, preferred_element_type=jnp.float32)

    @pl.when(i == 0)
    def _():
        hg_ref[...] = part

    @pl.when(i > 0)
    def _():
        hg_ref[...] = hg_ref[...] + part

    @pl.when(i == (N // BN) - 1)
    def _():
        z = jnp.dot(hg_ref[...], wh_ref[...], preferred_element_type=jnp.float32) + bh_ref[0:1, :]
        act = z * jax.nn.sigmoid(z)
        out_ref[...] = jnp.dot(act, wo_ref[...], preferred_element_type=jnp.float32) + bo_ref[0:1, :]


def _final(h, hu, agg, nrm, b3, wh, bh, wo, bo):
    out_shape = (
        jax.ShapeDtypeStruct((N, D), jnp.float32),
        jax.ShapeDtypeStruct((G, D), jnp.float32),
    )
    nb = pl.BlockSpec((BN, D), lambda i: (i, 0))
    return pl.pallas_call(
        _final_body,
        grid=(N // BN,),
        in_specs=[nb, nb, nb, nb,
                  pl.BlockSpec((1, 1, BN), lambda i: (i, 0, 0)),
                  _wspec((D, D)), _wspec((8, D)), _wspec((D, D)), _wspec((8, D))],
        out_specs=[nb, pl.BlockSpec((G, D), lambda i: (0, 0))],
        out_shape=out_shape,
        scratch_shapes=[pltpu.VMEM((G, D), jnp.float32)],
    )(h, hu, agg, nrm, b3, wh, bh, wo, bo)


def _gather(tabd, tabs, src, dst):
    mesh = plsc.VectorSubcoreMesh(core_axis_name="c", subcore_axis_name="s")

    @functools.partial(
        pl.kernel,
        out_type=(
            jax.ShapeDtypeStruct((E, D), jnp.float32),
            jax.ShapeDtypeStruct((E, 2 * D), jnp.float32),
        ),
        mesh=mesh,
        scratch_types=[
            pltpu.VMEM((CH,), jnp.int32),
            pltpu.VMEM((CH,), jnp.int32),
            pltpu.VMEM((CH, D), jnp.float32),
            pltpu.VMEM((CH, 2 * D), jnp.float32),
            pltpu.SemaphoreType.DMA,
            pltpu.SemaphoreType.DMA,
        ],
    )
    def k(tabd_h, tabs_h, src_h, dst_h, gd_h, gs_h,
          idxd, idxs, rowsd, rowss, semd, sems):
        wid = lax.axis_index("s") * NC + lax.axis_index("c")
        base0 = wid * EPW

        def body(j, carry):
            base = base0 + j * CH
            pltpu.sync_copy(dst_h.at[pl.ds(base, CH)], idxd)
            pltpu.sync_copy(src_h.at[pl.ds(base, CH)], idxs)
            cd = pltpu.async_copy(tabd_h.at[idxd], rowsd, semd)
            cs = pltpu.async_copy(tabs_h.at[idxs], rowss, sems)
            cd.wait()
            cs.wait()
            pltpu.sync_copy(rowsd, gd_h.at[pl.ds(base, CH)])
            pltpu.sync_copy(rowss, gs_h.at[pl.ds(base, CH)])
            return carry

        lax.fori_loop(0, EPW // CH, body, 0)

    return k(tabd, tabs, src, dst)


def _scatter(msg, eta, dst):
    mesh = plsc.VectorSubcoreMesh(core_axis_name="c", subcore_axis_name="s")

    @functools.partial(
        pl.kernel,
        out_type=(
            jax.ShapeDtypeStruct((N, D), jnp.float32),
            jax.ShapeDtypeStruct((N, D), jnp.float32),
        ),
        mesh=mesh,
        scratch_types=[
            pltpu.VMEM((CH,), jnp.int32),
            pltpu.VMEM((CH, D), jnp.float32),
            pltpu.VMEM((DRN, D), jnp.float32),
            pltpu.VMEM_SHARED((N, D), jnp.float32),
        ],
    )
    def k(msg_h, eta_h, dst_h, agg_h, nrm_h, idx, rows, buf, acc):
        c = lax.axis_index("c")
        s = lax.axis_index("s")

        def zrow(i, carry):
            for t in range(D // 16):
                buf[i, pl.ds(t * 16, 16)] = jnp.zeros((16,), jnp.float32)
            return carry

        lax.fori_loop(0, DRN, zrow, 0)

        def zacc(j, carry):
            t = s + NS * j

            @pl.when(t < NCHK)
            def _():
                pltpu.sync_copy(buf, acc.at[pl.ds(t * DRN, DRN)])

            return carry

        lax.fori_loop(0, (NCHK + NS - 1) // NS, zacc, 0)
        plsc.subcore_barrier()

        base0 = s * EPS

        def chunk(j, carry):
            base = base0 + j * CH
            pltpu.sync_copy(dst_h.at[pl.ds(base, CH)], idx)

            @pl.when(c == 0)
            def _():
                pltpu.sync_copy(msg_h.at[pl.ds(base, CH)], rows)

            @pl.when(c == 1)
            def _():
                pltpu.sync_copy(eta_h.at[pl.ds(base, CH)], rows)

            pltpu.sync_copy(rows, acc.at[idx], add=True)
            return carry

        lax.fori_loop(0, EPS // CH, chunk, 0)
        plsc.subcore_barrier()

        def drain(j, carry):
            t = s + NS * j

            @pl.when(t < NCHK)
            def _():
                off = t * DRN
                pltpu.sync_copy(acc.at[pl.ds(off, DRN)], buf)

                @pl.when(c == 0)
                def _():
                    pltpu.sync_copy(buf, agg_h.at[pl.ds(off, DRN)])

                @pl.when(c == 1)
                def _():
                    pltpu.sync_copy(buf, nrm_h.at[pl.ds(off, DRN)])

            return carry

        lax.fori_loop(0, (NCHK + NS - 1) // NS, drain, 0)

    return k(msg, eta, dst)


def kernel(x_bnd, params, edge_index, x_atm, x_atm_batch):
    src = edge_index[0]
    dst = edge_index[1]
    x2 = x_bnd.reshape(E, 1)
    xa2 = x_atm.reshape(N, 1)
    b3 = x_atm_batch.reshape(N // BN, 1, BN)
    emb = jnp.pad(params["embed"], ((0, 5), (0, 0)))
    wo = jnp.pad(params["W_out"], ((0, 0), (0, D - 3)))
    bo = jnp.pad(params["b_out"], (0, D - 3)).reshape(1, D)
    bh = jnp.pad(params["b_head"].reshape(1, D), ((0, 7), (0, 0)))
    bc_all = [jnp.pad(params["bC"][l].reshape(1, D), ((0, 7), (0, 0)))
              for l in range(LAYERS)]
    bias_all = [jnp.pad(jnp.stack([params["bA"][l], params["bB"][l],
                                   params["bV"][l], params["bU"][l]]),
                        ((0, 4), (0, 0)))
                for l in range(LAYERS)]

    h, tabd, tabs, hu = _node_pre(xa2, emb, params["A"][0], params["B"][0],
                                  params["V"][0], params["U"][0], bias_all[0])
    e = x2
    agg = nrm = None
    for l in range(LAYERS):
        if l > 0:
            h, tabd, tabs, hu = _node_mid(h, hu, agg, nrm, params["A"][l],
                                          params["B"][l], params["V"][l],
                                          params["U"][l], bias_all[l])
        gd, gs = _gather(tabd, tabs, src, dst)
        e, msg, eta = _edge(l == 0, e, gd, gs, params["C"][l], bc_all[l])
        agg, nrm = _scatter(msg, eta, dst)

    h_out, out_pad = _final(h, hu, agg, nrm, b3, params["W_head"], bh, wo, bo)
    return out_pad[:, :3], h_out


# speedup vs baseline: 3.9575x; 1.4573x over previous
import functools

import jax
import jax.numpy as jnp
from jax import lax
from jax.experimental import pallas as pl
from jax.experimental.pallas import tpu as pltpu
from jax.experimental.pallas import tpu_sc as plsc

N = 10000
E = 320000
D = 128
G = 64
LAYERS = 6
CUT = 6.0

BN = 1000
BE = 2000

NC = 2
NS = 16
NW = NC * NS
EPW = E // NW
EPS = E // NS
CH = 80
DRN = 200
NCHK = N // DRN


def _node_pre_body(xa_ref, emb_ref, a_ref, b_ref, v_ref, u_ref, bias_ref,
                   h_ref, tabd_ref, tabs_ref, hu_ref):
    xa = xa_ref[...]
    e0 = emb_ref[0:1, :]
    e1 = emb_ref[1:2, :]
    e2 = emb_ref[2:3, :]
    h = jnp.where(xa == 0, e0, jnp.where(xa == 1, e1, e2))
    h_ref[...] = h
    tabd_ref[...] = jnp.dot(h, a_ref[...], preferred_element_type=jnp.float32) + bias_ref[0:1, :]
    tabs_ref[:, 0:D] = jnp.dot(h, b_ref[...], preferred_element_type=jnp.float32) + bias_ref[1:2, :]
    tabs_ref[:, D:2 * D] = jnp.dot(h, v_ref[...], preferred_element_type=jnp.float32) + bias_ref[2:3, :]
    hu_ref[...] = jnp.dot(h, u_ref[...], preferred_element_type=jnp.float32) + bias_ref[3:4, :]


def _node_mid_body(h_ref, hu_ref, agg_ref, nrm_ref, a_ref, b_ref, v_ref,
                   u_ref, bias_ref, hn_ref, tabd_ref, tabs_ref, hun_ref):
    upd = hu_ref[...] + agg_ref[...] / (nrm_ref[...] + 1e-6)
    h = h_ref[...] + upd * jax.nn.sigmoid(upd)
    hn_ref[...] = h
    tabd_ref[...] = jnp.dot(h, a_ref[...], preferred_element_type=jnp.float32) + bias_ref[0:1, :]
    tabs_ref[:, 0:D] = jnp.dot(h, b_ref[...], preferred_element_type=jnp.float32) + bias_ref[1:2, :]
    tabs_ref[:, D:2 * D] = jnp.dot(h, v_ref[...], preferred_element_type=jnp.float32) + bias_ref[2:3, :]
    hun_ref[...] = jnp.dot(h, u_ref[...], preferred_element_type=jnp.float32) + bias_ref[3:4, :]


def _wspec(shape):
    return pl.BlockSpec(shape, lambda i: (0,) * len(shape))


def _node_pre(xa2, emb, a, b, v, u, bias):
    out_shape = (
        jax.ShapeDtypeStruct((N, D), jnp.float32),
        jax.ShapeDtypeStruct((N, D), jnp.float32),
        jax.ShapeDtypeStruct((N, 2 * D), jnp.float32),
        jax.ShapeDtypeStruct((N, D), jnp.float32),
    )
    return pl.pallas_call(
        _node_pre_body,
        grid=(N // BN,),
        in_specs=[
            pl.BlockSpec((BN, 1), lambda i: (i, 0)),
            _wspec((8, D)), _wspec((D, D)), _wspec((D, D)), _wspec((D, D)),
            _wspec((D, D)), _wspec((8, D)),
        ],
        out_specs=[
            pl.BlockSpec((BN, D), lambda i: (i, 0)),
            pl.BlockSpec((BN, D), lambda i: (i, 0)),
            pl.BlockSpec((BN, 2 * D), lambda i: (i, 0)),
            pl.BlockSpec((BN, D), lambda i: (i, 0)),
        ],
        out_shape=out_shape,
    )(xa2, emb, a, b, v, u, bias)


def _node_mid(h, hu, agg, nrm, a, b, v, u, bias):
    out_shape = (
        jax.ShapeDtypeStruct((N, D), jnp.float32),
        jax.ShapeDtypeStruct((N, D), jnp.float32),
        jax.ShapeDtypeStruct((N, 2 * D), jnp.float32),
        jax.ShapeDtypeStruct((N, D), jnp.float32),
    )
    nb = pl.BlockSpec((BN, D), lambda i: (i, 0))
    return pl.pallas_call(
        _node_mid_body,
        grid=(N // BN,),
        in_specs=[nb, nb, nb, nb,
                  _wspec((D, D)), _wspec((D, D)), _wspec((D, D)),
                  _wspec((D, D)), _wspec((8, D))],
        out_specs=[nb, nb, pl.BlockSpec((BN, 2 * D), lambda i: (i, 0)), nb],
        out_shape=out_shape,
    )(h, hu, agg, nrm, a, b, v, u, bias)


def _edge_body(first, e_ref, gd_ref, gs_ref, c_ref, bc_ref,
               en_ref, msg_ref, eta_ref):
    if first:
        x = e_ref[...]
        n = lax.broadcasted_iota(jnp.int32, (BE, D), 1).astype(jnp.float32) + 1.0
        e = jnp.sqrt(2.0 / CUT) * jnp.sin(n * (jnp.pi / CUT) * x) / (x + 1e-9)
    else:
        e = e_ref[...]
    gs = gs_ref[...]
    ep = (gd_ref[...] + gs[:, 0:D] + bc_ref[0:1, :]
          + jnp.dot(e, c_ref[...], preferred_element_type=jnp.float32))
    eta = jax.nn.sigmoid(ep)
    en_ref[...] = e + ep * eta
    msg_ref[...] = eta * gs[:, D:2 * D]
    eta_ref[...] = eta


def _edge(first, e_in, gd, gs, c, bc):
    out_shape = (
        jax.ShapeDtypeStruct((E, D), jnp.float32),
        jax.ShapeDtypeStruct((E, D), jnp.float32),
        jax.ShapeDtypeStruct((E, D), jnp.float32),
    )
    eb = pl.BlockSpec((BE, D), lambda i: (i, 0))
    e_spec = pl.BlockSpec((BE, 1), lambda i: (i, 0)) if first else eb
    return pl.pallas_call(
        functools.partial(_edge_body, first),
        grid=(E // BE,),
        in_specs=[e_spec, eb, pl.BlockSpec((BE, 2 * D), lambda i: (i, 0)),
                  _wspec((D, D)), _wspec((8, D))],
        out_specs=[eb, eb, eb],
        out_shape=out_shape,
    )(e_in, gd, gs, c, bc)


def _final_body(h_ref, hu_ref, agg_ref, nrm_ref, b3_ref, wh_ref, bh_ref,
                wo_ref, bo_ref, hout_ref, out_ref, hg_ref):
    i = pl.program_id(0)
    upd = hu_ref[...] + agg_ref[...] / (nrm_ref[...] + 1e-6)
    h = h_ref[...] + upd * jax.nn.sigmoid(upd)
    hout_ref[...] = h
    bids = jnp.reshape(b3_ref[...], (1, BN))
    oh = (lax.broadcasted_iota(jnp.int32, (G, BN), 0) == bids).astype(jnp.float32)
    part = jnp.dot(oh, h, preferred_element_type=jnp.float32)

    @pl.when(i == 0)
    def _():
        hg_ref[...] = part

    @pl.when(i > 0)
    def _():
        hg_ref[...] = hg_ref[...] + part

    @pl.when(i == (N // BN) - 1)
    def _():
        z = jnp.dot(hg_ref[...], wh_ref[...], preferred_element_type=jnp.float32) + bh_ref[0:1, :]
        act = z * jax.nn.sigmoid(z)
        out_ref[...] = jnp.dot(act, wo_ref[...], preferred_element_type=jnp.float32) + bo_ref[0:1, :]


def _final(h, hu, agg, nrm, b3, wh, bh, wo, bo):
    out_shape = (
        jax.ShapeDtypeStruct((N, D), jnp.float32),
        jax.ShapeDtypeStruct((G, D), jnp.float32),
    )
    nb = pl.BlockSpec((BN, D), lambda i: (i, 0))
    return pl.pallas_call(
        _final_body,
        grid=(N // BN,),
        in_specs=[nb, nb, nb, nb,
                  pl.BlockSpec((1, 1, BN), lambda i: (i, 0, 0)),
                  _wspec((D, D)), _wspec((8, D)), _wspec((D, D)), _wspec((8, D))],
        out_specs=[nb, pl.BlockSpec((G, D), lambda i: (0, 0))],
        out_shape=out_shape,
        scratch_shapes=[pltpu.VMEM((G, D), jnp.float32)],
    )(h, hu, agg, nrm, b3, wh, bh, wo, bo)


def _gather(tabd, tabs, src, dst):
    mesh = plsc.VectorSubcoreMesh(core_axis_name="c", subcore_axis_name="s")
    NCH = EPW // CH

    @functools.partial(
        pl.kernel,
        out_type=(
            jax.ShapeDtypeStruct((E, D), jnp.float32),
            jax.ShapeDtypeStruct((E, 2 * D), jnp.float32),
        ),
        mesh=mesh,
        scratch_types=[
            pltpu.VMEM((EPW,), jnp.int32),
            pltpu.VMEM((EPW,), jnp.int32),
            pltpu.VMEM((CH, D), jnp.float32),
            pltpu.VMEM((CH, D), jnp.float32),
            pltpu.VMEM((CH, 2 * D), jnp.float32),
            pltpu.VMEM((CH, 2 * D), jnp.float32),
            pltpu.SemaphoreType.DMA,
            pltpu.SemaphoreType.DMA,
            pltpu.SemaphoreType.DMA,
            pltpu.SemaphoreType.DMA,
        ],
    )
    def k(tabd_h, tabs_h, src_h, dst_h, gd_h, gs_h,
          idxd, idxs, rd0, rd1, rs0, rs1, sd0, sd1, ss0, ss1):
        wid = lax.axis_index("s") * NC + lax.axis_index("c")
        base0 = wid * EPW
        pltpu.sync_copy(dst_h.at[pl.ds(base0, EPW)], idxd)
        pltpu.sync_copy(src_h.at[pl.ds(base0, EPW)], idxs)

        def fire(j, rd, rs, sd, ss):
            pltpu.async_copy(tabd_h.at[idxd.at[pl.ds(j * CH, CH)]], rd, sd)
            pltpu.async_copy(tabs_h.at[idxs.at[pl.ds(j * CH, CH)]], rs, ss)

        def drain_store(j, rd, rs, sd, ss):
            pltpu.make_async_copy(tabd_h.at[idxd.at[pl.ds(j * CH, CH)]], rd, sd).wait()
            pltpu.make_async_copy(tabs_h.at[idxs.at[pl.ds(j * CH, CH)]], rs, ss).wait()
            base = base0 + j * CH
            pltpu.sync_copy(rd, gd_h.at[pl.ds(base, CH)])
            pltpu.sync_copy(rs, gs_h.at[pl.ds(base, CH)])

        fire(0, rd0, rs0, sd0, ss0)

        def body(kk, carry):
            a = 2 * kk
            fire(a + 1, rd1, rs1, sd1, ss1)
            drain_store(a, rd0, rs0, sd0, ss0)
            fire(a + 2, rd0, rs0, sd0, ss0)
            drain_store(a + 1, rd1, rs1, sd1, ss1)
            return carry

        lax.fori_loop(0, (NCH - 1) // 2, body, 0)
        drain_store(NCH - 1, rd0, rs0, sd0, ss0)

    return k(tabd, tabs, src, dst)


def _scatter(msg, eta, dst):
    mesh = plsc.VectorSubcoreMesh(core_axis_name="c", subcore_axis_name="s")

    @functools.partial(
        pl.kernel,
        out_type=(
            jax.ShapeDtypeStruct((N, D), jnp.float32),
            jax.ShapeDtypeStruct((N, D), jnp.float32),
        ),
        mesh=mesh,
        scratch_types=[
            pltpu.VMEM((CH,), jnp.int32),
            pltpu.VMEM((CH,), jnp.int32),
            pltpu.VMEM((CH, D), jnp.float32),
            pltpu.VMEM((CH, D), jnp.float32),
            pltpu.VMEM((DRN, D), jnp.float32),
            pltpu.VMEM_SHARED((N, D), jnp.float32),
            pltpu.SemaphoreType.DMA,
            pltpu.SemaphoreType.DMA,
            pltpu.SemaphoreType.DMA,
            pltpu.SemaphoreType.DMA,
        ],
    )
    def k(msg_h, eta_h, dst_h, agg_h, nrm_h, idx0, idx1, rows0, rows1,
          buf, acc, si0, si1, sr0, sr1):
        c = lax.axis_index("c")
        s = lax.axis_index("s")

        def zrow(i, carry):
            for t in range(D // 16):
                buf[i, pl.ds(t * 16, 16)] = jnp.zeros((16,), jnp.float32)
            return carry

        lax.fori_loop(0, DRN, zrow, 0)

        def zacc(j, carry):
            t = s + NS * j

            @pl.when(t < NCHK)
            def _():
                pltpu.sync_copy(buf, acc.at[pl.ds(t * DRN, DRN)])

            return carry

        lax.fori_loop(0, (NCHK + NS - 1) // NS, zacc, 0)
        plsc.subcore_barrier()

        base0 = s * EPS
        NCH = EPS // CH

        def fire(j, idx, rows, si, sr):
            base = base0 + j * CH
            pltpu.async_copy(dst_h.at[pl.ds(base, CH)], idx, si)

            @pl.when(c == 0)
            def _():
                pltpu.async_copy(msg_h.at[pl.ds(base, CH)], rows, sr)

            @pl.when(c == 1)
            def _():
                pltpu.async_copy(eta_h.at[pl.ds(base, CH)], rows, sr)

        def drain_scatter(j, idx, rows, si, sr):
            base = base0 + j * CH
            pltpu.make_async_copy(dst_h.at[pl.ds(base, CH)], idx, si).wait()
            pltpu.make_async_copy(msg_h.at[pl.ds(base, CH)], rows, sr).wait()
            pltpu.sync_copy(rows, acc.at[idx], add=True)

        fire(0, idx0, rows0, si0, sr0)

        def chunk(kk, carry):
            a = 2 * kk
            fire(a + 1, idx1, rows1, si1, sr1)
            drain_scatter(a, idx0, rows0, si0, sr0)

            @pl.when(a + 2 < NCH)
            def _():
                fire(a + 2, idx0, rows0, si0, sr0)

            drain_scatter(a + 1, idx1, rows1, si1, sr1)
            return carry

        lax.fori_loop(0, NCH // 2, chunk, 0)
        plsc.subcore_barrier()

        def drain(j, carry):
            t = s + NS * j

            @pl.when(t < NCHK)
            def _():
                off = t * DRN
                pltpu.sync_copy(acc.at[pl.ds(off, DRN)], buf)

                @pl.when(c == 0)
                def _():
                    pltpu.sync_copy(buf, agg_h.at[pl.ds(off, DRN)])

                @pl.when(c == 1)
                def _():
                    pltpu.sync_copy(buf, nrm_h.at[pl.ds(off, DRN)])

            return carry

        lax.fori_loop(0, (NCHK + NS - 1) // NS, drain, 0)

    return k(msg, eta, dst)


def kernel(x_bnd, params, edge_index, x_atm, x_atm_batch):
    src = edge_index[0]
    dst = edge_index[1]
    x2 = x_bnd.reshape(E, 1)
    xa2 = x_atm.reshape(N, 1)
    b3 = x_atm_batch.reshape(N // BN, 1, BN)
    emb = jnp.pad(params["embed"], ((0, 5), (0, 0)))
    wo = jnp.pad(params["W_out"], ((0, 0), (0, D - 3)))
    bo = jnp.pad(params["b_out"], (0, D - 3)).reshape(1, D)
    bh = jnp.pad(params["b_head"].reshape(1, D), ((0, 7), (0, 0)))
    bc_all = [jnp.pad(params["bC"][l].reshape(1, D), ((0, 7), (0, 0)))
              for l in range(LAYERS)]
    bias_all = [jnp.pad(jnp.stack([params["bA"][l], params["bB"][l],
                                   params["bV"][l], params["bU"][l]]),
                        ((0, 4), (0, 0)))
                for l in range(LAYERS)]

    h, tabd, tabs, hu = _node_pre(xa2, emb, params["A"][0], params["B"][0],
                                  params["V"][0], params["U"][0], bias_all[0])
    e = x2
    agg = nrm = None
    for l in range(LAYERS):
        if l > 0:
            h, tabd, tabs, hu = _node_mid(h, hu, agg, nrm, params["A"][l],
                                          params["B"][l], params["V"][l],
                                          params["U"][l], bias_all[l])
        gd, gs = _gather(tabd, tabs, src, dst)
        e, msg, eta = _edge(l == 0, e, gd, gs, params["C"][l], bc_all[l])
        agg, nrm = _scatter(msg, eta, dst)

    h_out, out_pad = _final(h, hu, agg, nrm, b3, params["W_head"], bh, wo, bo)
    return out_pad[:, :3], h_out


# speedup vs baseline: 4.6066x; 1.1640x over previous
import functools

import jax
import jax.numpy as jnp
from jax import lax
from jax.experimental import pallas as pl
from jax.experimental.pallas import tpu as pltpu
from jax.experimental.pallas import tpu_sc as plsc

N = 10000
E = 320000
D = 128
G = 64
LAYERS = 6
CUT = 6.0

BN = 1000
BE = 2000

NC = 2
NS = 16
NW = NC * NS
EPW = E // NW
EPS = E // NS
CH = 80
DRN = 200
NCHK = N // DRN


def _pack2(a, b):
    ab = lax.bitcast_convert_type(a.astype(jnp.bfloat16), jnp.uint16).astype(jnp.uint32)
    bb = lax.bitcast_convert_type(b.astype(jnp.bfloat16), jnp.uint16).astype(jnp.uint32)
    return (ab << 16) | bb


def _unpack2(u):
    a = lax.bitcast_convert_type((u >> 16).astype(jnp.uint16), jnp.bfloat16)
    b = lax.bitcast_convert_type((u & 0xFFFF).astype(jnp.uint16), jnp.bfloat16)
    return a.astype(jnp.float32), b.astype(jnp.float32)


def _node_pre_body(xa_ref, emb_ref, a_ref, b_ref, v_ref, u_ref, bias_ref,
                   h_ref, tabd_ref, tabs_ref, hu_ref):
    xa = xa_ref[...]
    e0 = emb_ref[0:1, :]
    e1 = emb_ref[1:2, :]
    e2 = emb_ref[2:3, :]
    h = jnp.where(xa == 0, e0, jnp.where(xa == 1, e1, e2))
    h_ref[...] = h
    tabd_ref[...] = jnp.dot(h, a_ref[...], preferred_element_type=jnp.float32) + bias_ref[0:1, :]
    tabs_ref[...] = _pack2(
        jnp.dot(h, b_ref[...], preferred_element_type=jnp.float32) + bias_ref[1:2, :],
        jnp.dot(h, v_ref[...], preferred_element_type=jnp.float32) + bias_ref[2:3, :])
    hu_ref[...] = jnp.dot(h, u_ref[...], preferred_element_type=jnp.float32) + bias_ref[3:4, :]


def _node_mid_body(h_ref, hu_ref, agg_ref, nrm_ref, a_ref, b_ref, v_ref,
                   u_ref, bias_ref, hn_ref, tabd_ref, tabs_ref, hun_ref):
    upd = hu_ref[...] + agg_ref[...] / (nrm_ref[...] + 1e-6)
    h = h_ref[...] + upd * jax.nn.sigmoid(upd)
    hn_ref[...] = h
    tabd_ref[...] = jnp.dot(h, a_ref[...], preferred_element_type=jnp.float32) + bias_ref[0:1, :]
    tabs_ref[...] = _pack2(
        jnp.dot(h, b_ref[...], preferred_element_type=jnp.float32) + bias_ref[1:2, :],
        jnp.dot(h, v_ref[...], preferred_element_type=jnp.float32) + bias_ref[2:3, :])
    hun_ref[...] = jnp.dot(h, u_ref[...], preferred_element_type=jnp.float32) + bias_ref[3:4, :]


def _wspec(shape):
    return pl.BlockSpec(shape, lambda i: (0,) * len(shape))


def _node_pre(xa2, emb, a, b, v, u, bias):
    out_shape = (
        jax.ShapeDtypeStruct((N, D), jnp.float32),
        jax.ShapeDtypeStruct((N, D), jnp.float32),
        jax.ShapeDtypeStruct((N, D), jnp.uint32),
        jax.ShapeDtypeStruct((N, D), jnp.float32),
    )
    return pl.pallas_call(
        _node_pre_body,
        grid=(N // BN,),
        in_specs=[
            pl.BlockSpec((BN, 1), lambda i: (i, 0)),
            _wspec((8, D)), _wspec((D, D)), _wspec((D, D)), _wspec((D, D)),
            _wspec((D, D)), _wspec((8, D)),
        ],
        out_specs=[
            pl.BlockSpec((BN, D), lambda i: (i, 0)),
            pl.BlockSpec((BN, D), lambda i: (i, 0)),
            pl.BlockSpec((BN, D), lambda i: (i, 0)),
            pl.BlockSpec((BN, D), lambda i: (i, 0)),
        ],
        out_shape=out_shape,
    )(xa2, emb, a, b, v, u, bias)


def _node_mid(h, hu, agg, nrm, a, b, v, u, bias):
    out_shape = (
        jax.ShapeDtypeStruct((N, D), jnp.float32),
        jax.ShapeDtypeStruct((N, D), jnp.float32),
        jax.ShapeDtypeStruct((N, D), jnp.uint32),
        jax.ShapeDtypeStruct((N, D), jnp.float32),
    )
    nb = pl.BlockSpec((BN, D), lambda i: (i, 0))
    return pl.pallas_call(
        _node_mid_body,
        grid=(N // BN,),
        in_specs=[nb, nb, nb, nb,
                  _wspec((D, D)), _wspec((D, D)), _wspec((D, D)),
                  _wspec((D, D)), _wspec((8, D))],
        out_specs=[nb, nb, pl.BlockSpec((BN, D), lambda i: (i, 0)), nb],
        out_shape=out_shape,
    )(h, hu, agg, nrm, a, b, v, u, bias)


def _edge_body(first, e_ref, gd_ref, gs_ref, c_ref, bc_ref,
               en_ref, msg_ref, eta_ref):
    if first:
        x = e_ref[...]
        n = lax.broadcasted_iota(jnp.int32, (BE, D), 1).astype(jnp.float32) + 1.0
        e = jnp.sqrt(2.0 / CUT) * jnp.sin(n * (jnp.pi / CUT) * x) / (x + 1e-9)
    else:
        e = e_ref[...]
    gb, gv = _unpack2(gs_ref[...])
    ep = (gd_ref[...] + gb + bc_ref[0:1, :]
          + jnp.dot(e, c_ref[...], preferred_element_type=jnp.float32))
    eta = jax.nn.sigmoid(ep)
    en_ref[...] = e + ep * eta
    msg_ref[...] = eta * gv
    eta_ref[...] = eta


def _edge(first, e_in, gd, gs, c, bc):
    out_shape = (
        jax.ShapeDtypeStruct((E, D), jnp.float32),
        jax.ShapeDtypeStruct((E, D), jnp.float32),
        jax.ShapeDtypeStruct((E, D), jnp.float32),
    )
    eb = pl.BlockSpec((BE, D), lambda i: (i, 0))
    e_spec = pl.BlockSpec((BE, 1), lambda i: (i, 0)) if first else eb
    return pl.pallas_call(
        functools.partial(_edge_body, first),
        grid=(E // BE,),
        in_specs=[e_spec, eb, pl.BlockSpec((BE, D), lambda i: (i, 0)),
                  _wspec((D, D)), _wspec((8, D))],
        out_specs=[eb, eb, eb],
        out_shape=out_shape,
    )(e_in, gd, gs, c, bc)


def _final_body(h_ref, hu_ref, agg_ref, nrm_ref, b3_ref, wh_ref, bh_ref,
                wo_ref, bo_ref, hout_ref, out_ref, hg_ref):
    i = pl.program_id(0)
    upd = hu_ref[...] + agg_ref[...] / (nrm_ref[...] + 1e-6)
    h = h_ref[...] + upd * jax.nn.sigmoid(upd)
    hout_ref[...] = h
    bids = jnp.reshape(b3_ref[...], (1, BN))
    oh = (lax.broadcasted_iota(jnp.int32, (G, BN), 0) == bids).astype(jnp.float32)
    part = jnp.dot(oh, h, preferred_element_type=jnp.float32)

    @pl.when(i == 0)
    def _():
        hg_ref[...] = part

    @pl.when(i > 0)
    def _():
        hg_ref[...] = hg_ref[...] + part

    @pl.when(i == (N // BN) - 1)
    def _():
        z = jnp.dot(hg_ref[...], wh_ref[...], preferred_element_type=jnp.float32) + bh_ref[0:1, :]
        act = z * jax.nn.sigmoid(z)
        out_ref[...] = jnp.dot(act, wo_ref[...], preferred_element_type=jnp.float32) + bo_ref[0:1, :]


def _final(h, hu, agg, nrm, b3, wh, bh, wo, bo):
    out_shape = (
        jax.ShapeDtypeStruct((N, D), jnp.float32),
        jax.ShapeDtypeStruct((G, D), jnp.float32),
    )
    nb = pl.BlockSpec((BN, D), lambda i: (i, 0))
    return pl.pallas_call(
        _final_body,
        grid=(N // BN,),
        in_specs=[nb, nb, nb, nb,
                  pl.BlockSpec((1, 1, BN), lambda i: (i, 0, 0)),
                  _wspec((D, D)), _wspec((8, D)), _wspec((D, D)), _wspec((8, D))],
        out_specs=[nb, pl.BlockSpec((G, D), lambda i: (0, 0))],
        out_shape=out_shape,
        scratch_shapes=[pltpu.VMEM((G, D), jnp.float32)],
    )(h, hu, agg, nrm, b3, wh, bh, wo, bo)


def _gather(tabd, tabs, src, dst):
    mesh = plsc.VectorSubcoreMesh(core_axis_name="c", subcore_axis_name="s")
    NCH = EPW // CH

    @functools.partial(
        pl.kernel,
        out_type=(
            jax.ShapeDtypeStruct((E, D), jnp.float32),
            jax.ShapeDtypeStruct((E, D), jnp.uint32),
        ),
        mesh=mesh,
        scratch_types=[
            pltpu.VMEM((EPW,), jnp.int32),
            pltpu.VMEM((EPW,), jnp.int32),
            pltpu.VMEM((CH, D), jnp.float32),
            pltpu.VMEM((CH, D), jnp.float32),
            pltpu.VMEM((CH, D), jnp.uint32),
            pltpu.VMEM((CH, D), jnp.uint32),
            pltpu.SemaphoreType.DMA,
            pltpu.SemaphoreType.DMA,
            pltpu.SemaphoreType.DMA,
            pltpu.SemaphoreType.DMA,
        ],
    )
    def k(tabd_h, tabs_h, src_h, dst_h, gd_h, gs_h,
          idxd, idxs, rd0, rd1, rs0, rs1, sd0, sd1, ss0, ss1):
        wid = lax.axis_index("s") * NC + lax.axis_index("c")
        base0 = wid * EPW
        pltpu.sync_copy(dst_h.at[pl.ds(base0, EPW)], idxd)
        pltpu.sync_copy(src_h.at[pl.ds(base0, EPW)], idxs)

        def fire(j, rd, rs, sd, ss):
            pltpu.async_copy(tabd_h.at[idxd.at[pl.ds(j * CH, CH)]], rd, sd)
            pltpu.async_copy(tabs_h.at[idxs.at[pl.ds(j * CH, CH)]], rs, ss)

        def drain_store(j, rd, rs, sd, ss):
            pltpu.make_async_copy(tabd_h.at[idxd.at[pl.ds(j * CH, CH)]], rd, sd).wait()
            pltpu.make_async_copy(tabs_h.at[idxs.at[pl.ds(j * CH, CH)]], rs, ss).wait()
            base = base0 + j * CH
            pltpu.sync_copy(rd, gd_h.at[pl.ds(base, CH)])
            pltpu.sync_copy(rs, gs_h.at[pl.ds(base, CH)])

        fire(0, rd0, rs0, sd0, ss0)

        def body(kk, carry):
            a = 2 * kk
            fire(a + 1, rd1, rs1, sd1, ss1)
            drain_store(a, rd0, rs0, sd0, ss0)
            fire(a + 2, rd0, rs0, sd0, ss0)
            drain_store(a + 1, rd1, rs1, sd1, ss1)
            return carry

        lax.fori_loop(0, (NCH - 1) // 2, body, 0)
        drain_store(NCH - 1, rd0, rs0, sd0, ss0)

    return k(tabd, tabs, src, dst)


def _scatter(msg, eta, dst):
    mesh = plsc.VectorSubcoreMesh(core_axis_name="c", subcore_axis_name="s")

    @functools.partial(
        pl.kernel,
        out_type=(
            jax.ShapeDtypeStruct((N, D), jnp.float32),
            jax.ShapeDtypeStruct((N, D), jnp.float32),
        ),
        mesh=mesh,
        scratch_types=[
            pltpu.VMEM((CH,), jnp.int32),
            pltpu.VMEM((CH,), jnp.int32),
            pltpu.VMEM((CH, D), jnp.float32),
            pltpu.VMEM((CH, D), jnp.float32),
            pltpu.VMEM((DRN, D), jnp.float32),
            pltpu.VMEM_SHARED((N, D), jnp.float32),
            pltpu.SemaphoreType.DMA,
            pltpu.SemaphoreType.DMA,
            pltpu.SemaphoreType.DMA,
            pltpu.SemaphoreType.DMA,
        ],
    )
    def k(msg_h, eta_h, dst_h, agg_h, nrm_h, idx0, idx1, rows0, rows1,
          buf, acc, si0, si1, sr0, sr1):
        c = lax.axis_index("c")
        s = lax.axis_index("s")

        def zrow(i, carry):
            for t in range(D // 16):
                buf[i, pl.ds(t * 16, 16)] = jnp.zeros((16,), jnp.float32)
            return carry

        lax.fori_loop(0, DRN, zrow, 0)

        def zacc(j, carry):
            t = s + NS * j

            @pl.when(t < NCHK)
            def _():
                pltpu.sync_copy(buf, acc.at[pl.ds(t * DRN, DRN)])

            return carry

        lax.fori_loop(0, (NCHK + NS - 1) // NS, zacc, 0)
        plsc.subcore_barrier()

        base0 = s * EPS
        NCH = EPS // CH

        def fire(j, idx, rows, si, sr):
            base = base0 + j * CH
            pltpu.async_copy(dst_h.at[pl.ds(base, CH)], idx, si)

            @pl.when(c == 0)
            def _():
                pltpu.async_copy(msg_h.at[pl.ds(base, CH)], rows, sr)

            @pl.when(c == 1)
            def _():
                pltpu.async_copy(eta_h.at[pl.ds(base, CH)], rows, sr)

        def drain_scatter(j, idx, rows, si, sr):
            base = base0 + j * CH
            pltpu.make_async_copy(dst_h.at[pl.ds(base, CH)], idx, si).wait()
            pltpu.make_async_copy(msg_h.at[pl.ds(base, CH)], rows, sr).wait()
            pltpu.sync_copy(rows, acc.at[idx], add=True)

        fire(0, idx0, rows0, si0, sr0)

        def chunk(kk, carry):
            a = 2 * kk
            fire(a + 1, idx1, rows1, si1, sr1)
            drain_scatter(a, idx0, rows0, si0, sr0)

            @pl.when(a + 2 < NCH)
            def _():
                fire(a + 2, idx0, rows0, si0, sr0)

            drain_scatter(a + 1, idx1, rows1, si1, sr1)
            return carry

        lax.fori_loop(0, NCH // 2, chunk, 0)
        plsc.subcore_barrier()

        def drain(j, carry):
            t = s + NS * j

            @pl.when(t < NCHK)
            def _():
                off = t * DRN
                pltpu.sync_copy(acc.at[pl.ds(off, DRN)], buf)

                @pl.when(c == 0)
                def _():
                    pltpu.sync_copy(buf, agg_h.at[pl.ds(off, DRN)])

                @pl.when(c == 1)
                def _():
                    pltpu.sync_copy(buf, nrm_h.at[pl.ds(off, DRN)])

            return carry

        lax.fori_loop(0, (NCHK + NS - 1) // NS, drain, 0)

    return k(msg, eta, dst)


def kernel(x_bnd, params, edge_index, x_atm, x_atm_batch):
    src = edge_index[0]
    dst = edge_index[1]
    x2 = x_bnd.reshape(E, 1)
    xa2 = x_atm.reshape(N, 1)
    b3 = x_atm_batch.reshape(N // BN, 1, BN)
    emb = jnp.pad(params["embed"], ((0, 5), (0, 0)))
    wo = jnp.pad(params["W_out"], ((0, 0), (0, D - 3)))
    bo = jnp.pad(params["b_out"], (0, D - 3)).reshape(1, D)
    bh = jnp.pad(params["b_head"].reshape(1, D), ((0, 7), (0, 0)))
    bc_all = [jnp.pad(params["bC"][l].reshape(1, D), ((0, 7), (0, 0)))
              for l in range(LAYERS)]
    bias_all = [jnp.pad(jnp.stack([params["bA"][l], params["bB"][l],
                                   params["bV"][l], params["bU"][l]]),
                        ((0, 4), (0, 0)))
                for l in range(LAYERS)]

    h, tabd, tabs, hu = _node_pre(xa2, emb, params["A"][0], params["B"][0],
                                  params["V"][0], params["U"][0], bias_all[0])
    e = x2
    agg = nrm = None
    for l in range(LAYERS):
        if l > 0:
            h, tabd, tabs, hu = _node_mid(h, hu, agg, nrm, params["A"][l],
                                          params["B"][l], params["V"][l],
                                          params["U"][l], bias_all[l])
        gd, gs = _gather(tabd, tabs, src, dst)
        e, msg, eta = _edge(l == 0, e, gd, gs, params["C"][l], bc_all[l])
        agg, nrm = _scatter(msg, eta, dst)

    h_out, out_pad = _final(h, hu, agg, nrm, b3, params["W_head"], bh, wo, bo)
    return out_pad[:, :3], h_out
